# baseline (speedup 1.0000x reference)
import functools
import math

import jax
import jax.numpy as jnp
from jax import lax
from jax.experimental import pallas as pl
from jax.experimental.pallas import tpu as pltpu
from jax.experimental.pallas import tpu_sc as plsc

D_MODEL = 64
SCALE = math.sqrt(D_MODEL)
NUM_CORES = 2
NUM_SUBCORES = 16
NUM_WORKERS = NUM_CORES * NUM_SUBCORES
CHUNK = 512
GATHER = 128
LANES = 16


@functools.partial(jax.jit, static_argnames=("batch",))
def _emb_lookup(idx_flat, lut, *, batch):
    b_per_w = batch // NUM_WORKERS
    n_chunks = b_per_w // CHUNK
    mesh = plsc.VectorSubcoreMesh(core_axis_name="c", subcore_axis_name="s")

    @functools.partial(
        pl.kernel,
        out_type=jax.ShapeDtypeStruct((batch, D_MODEL), jnp.float32),
        mesh=mesh,
        scratch_types=[
            pltpu.VMEM((CHUNK,), jnp.int32),
            pltpu.VMEM((CHUNK, D_MODEL), jnp.float32),
            pltpu.SemaphoreType.DMA,
        ],
        compiler_params=pltpu.CompilerParams(use_tc_tiling_on_sc=False),
    )
    def body(lut_hbm, idx_hbm, out_hbm, idx_v, rows_v, sem):
        wid = lax.axis_index("s") * NUM_CORES + lax.axis_index("c")
        w_base = wid * b_per_w

        def chunk_step(g, _):
            base = w_base + g * CHUNK
            pltpu.sync_copy(idx_hbm.at[pl.ds(base, CHUNK)], idx_v)
            copies = [
                pltpu.async_copy(
                    lut_hbm.at[idx_v.at[pl.ds(j * GATHER, GATHER)]],
                    rows_v.at[pl.ds(j * GATHER, GATHER)],
                    sem,
                )
                for j in range(CHUNK // GATHER)
            ]
            for cp in copies:
                cp.wait()

            def scale_row(i, _):
                for j in range(D_MODEL // LANES):
                    sl = pl.ds(j * LANES, LANES)
                    rows_v[i, sl] = rows_v[i, sl] * SCALE
                return 0

            lax.fori_loop(0, CHUNK, scale_row, 0)
            pltpu.sync_copy(rows_v, out_hbm.at[pl.ds(base, CHUNK)])
            return 0

        lax.fori_loop(0, n_chunks, chunk_step, 0)

    return body(lut, idx_flat)


def kernel(x, lut):
    rows, cols = x.shape
    batch = rows * cols
    out = _emb_lookup(x.reshape(batch), lut, batch=batch)
    return out.reshape(rows, cols, D_MODEL)


# speedup vs baseline: 1.1296x; 1.1296x over previous
import functools
import math

import jax
import jax.numpy as jnp
from jax import lax
from jax.experimental import pallas as pl
from jax.experimental.pallas import tpu as pltpu
from jax.experimental.pallas import tpu_sc as plsc

D_MODEL = 64
SCALE = math.sqrt(D_MODEL)
NUM_CORES = 2
NUM_SUBCORES = 16
NUM_WORKERS = NUM_CORES * NUM_SUBCORES
CHUNK = 512
GATHER = 128
LANES = 16
N_GATHERS = CHUNK // GATHER


@functools.partial(jax.jit, static_argnames=("batch",))
def _emb_lookup(idx_flat, lut, *, batch):
    b_per_w = batch // NUM_WORKERS
    n_chunks = b_per_w // CHUNK
    n_pairs = n_chunks // 2
    mesh = plsc.VectorSubcoreMesh(core_axis_name="c", subcore_axis_name="s")

    @functools.partial(
        pl.kernel,
        out_type=jax.ShapeDtypeStruct((batch, D_MODEL), jnp.float32),
        mesh=mesh,
        scratch_types=[
            pltpu.VMEM((b_per_w,), jnp.int32),
            pltpu.VMEM((CHUNK, D_MODEL), jnp.float32),
            pltpu.VMEM((CHUNK, D_MODEL), jnp.float32),
            pltpu.SemaphoreType.DMA,
            pltpu.SemaphoreType.DMA,
            pltpu.SemaphoreType.DMA,
            pltpu.SemaphoreType.DMA,
        ],
        compiler_params=pltpu.CompilerParams(use_tc_tiling_on_sc=False),
    )
    def body(lut_hbm, idx_hbm, out_hbm, idx_all, rows0, rows1,
             sem_g0, sem_g1, sem_o0, sem_o1):
        wid = lax.axis_index("s") * NUM_CORES + lax.axis_index("c")
        w_base = wid * b_per_w

        def gather_copies(c, rows, sem):
            return [
                pltpu.make_async_copy(
                    lut_hbm.at[idx_all.at[pl.ds(c * CHUNK + j * GATHER, GATHER)]],
                    rows.at[pl.ds(j * GATHER, GATHER)],
                    sem,
                )
                for j in range(N_GATHERS)
            ]

        def fire_gathers(c, rows, sem):
            for cp in gather_copies(c, rows, sem):
                cp.start()

        def wait_gathers(c, rows, sem):
            for cp in gather_copies(c, rows, sem):
                cp.wait()

        def out_copy(c, rows, sem):
            return pltpu.make_async_copy(
                rows, out_hbm.at[pl.ds(w_base + c * CHUNK, CHUNK)], sem)

        def scale(rows):
            @plsc.parallel_loop(0, CHUNK, 1, unroll=8)
            def _(i):
                for j in range(D_MODEL // LANES):
                    sl = pl.ds(j * LANES, LANES)
                    rows[i, sl] = rows[i, sl] * SCALE

        pltpu.sync_copy(idx_hbm.at[pl.ds(w_base, b_per_w)], idx_all)
        fire_gathers(0, rows0, sem_g0)

        def pair(t, _):
            a = 2 * t

            @pl.when(t > 0)
            def _():
                out_copy(a - 1, rows1, sem_o1).wait()

            fire_gathers(a + 1, rows1, sem_g1)
            wait_gathers(a, rows0, sem_g0)
            scale(rows0)
            out_copy(a, rows0, sem_o0).start()

            wait_gathers(a + 1, rows1, sem_g1)
            scale(rows1)
            out_copy(a + 1, rows1, sem_o1).start()

            @pl.when(t < n_pairs - 1)
            def _():
                out_copy(a, rows0, sem_o0).wait()
                fire_gathers(a + 2, rows0, sem_g0)

            return 0

        lax.fori_loop(0, n_pairs, pair, 0)
        out_copy(n_chunks - 2, rows0, sem_o0).wait()
        out_copy(n_chunks - 1, rows1, sem_o1).wait()

    return body(lut, idx_flat)


def kernel(x, lut):
    rows, cols = x.shape
    batch = rows * cols
    out = _emb_lookup(x.reshape(batch), lut, batch=batch)
    return out.reshape(rows, cols, D_MODEL)
